# baseline (speedup 1.0000x reference)
import functools

import jax
import jax.numpy as jnp
from jax import lax
from jax.experimental import pallas as pl
from jax.experimental.pallas import tpu as pltpu
from jax.experimental.pallas import tpu_sc as plsc

N = 10000
C = 128
OC = 64
E = 320000

NC, NS = 2, 16
NW = NC * NS
BLK = 128
NB = 80
SBLK = 64
SNB = 160
SNBC = 8
SNPH = SNB // SNBC
EPW = NB * BLK
E_PAD = EPW * NW
NPAD = 10112
RPT = NPAD // NS
DEGW = 16

R = 632
GRID = NPAD // R


def _sc_mesh():
    return plsc.VectorSubcoreMesh(
        core_axis_name="c", subcore_axis_name="s",
        num_cores=NC, num_subcores=NS)


@functools.partial(
    pl.kernel,
    out_type=jax.ShapeDtypeStruct((NC, NPAD, DEGW), jnp.float32),
    mesh=_sc_mesh(),
    scratch_types=[
        pltpu.VMEM_SHARED((NPAD, DEGW), jnp.float32),
    ],
)
def _deg_kernel(dst_hbm, out_hbm, acc):
    cid = lax.axis_index("c")
    sid = lax.axis_index("s")
    wid = cid * NS + sid
    one = jnp.ones((16,), jnp.float32)
    zero = jnp.zeros((16,), jnp.float32)

    def body(dst_v, ones_v, zs_v):
        @pl.loop(0, BLK)
        def _(i):
            ones_v[i, :] = one

        @pl.loop(0, RPT)
        def _(i):
            zs_v[i, :] = zero

        base = sid * RPT
        pltpu.sync_copy(zs_v, acc.at[pl.ds(base, RPT)])
        pltpu.sync_copy(dst_hbm.at[wid], dst_v)
        plsc.subcore_barrier()

        @pl.loop(0, NB)
        def _(j):
            pltpu.sync_copy(ones_v, acc.at[dst_v.at[j]], add=True)

        plsc.subcore_barrier()
        pltpu.sync_copy(acc.at[pl.ds(base, RPT)],
                        out_hbm.at[cid, pl.ds(base, RPT)])

    pl.run_scoped(
        body,
        pltpu.VMEM((NB, BLK), jnp.int32),
        pltpu.VMEM((BLK, DEGW), jnp.float32),
        pltpu.VMEM((RPT, DEGW), jnp.float32),
    )


@functools.partial(
    pl.kernel,
    out_type=jax.ShapeDtypeStruct((NC, NPAD, C), jnp.float32),
    mesh=_sc_mesh(),
    scratch_types=[
        pltpu.VMEM_SHARED((NPAD, C), jnp.float32),
    ],
)
def _scatter_kernel(xs_hbm, src_hbm, dst_hbm, out_hbm, acc):
    cid = lax.axis_index("c")
    sid = lax.axis_index("s")
    wid = cid * NS + sid
    zero = jnp.zeros((16,), jnp.float32)
    base = sid * RPT

    def body(src_v, dst_v, rows_v, gs0, gs1, gs2, gs3, ss0, ss1, ss2, ss3):
        gsem = (gs0, gs1, gs2, gs3)
        ssem = (ss0, ss1, ss2, ss3)

        @pl.loop(0, SBLK)
        def _(i):
            for j in range(C // 16):
                rows_v[0, i, pl.ds(j * 16, 16)] = zero

        for t in range(RPT // SBLK):
            pltpu.sync_copy(rows_v.at[0], acc.at[pl.ds(base + t * SBLK, SBLK)])
        rem = RPT - (RPT // SBLK) * SBLK
        pltpu.sync_copy(rows_v.at[0, pl.ds(0, rem)],
                        acc.at[pl.ds(base + (RPT // SBLK) * SBLK, rem)])
        plsc.subcore_barrier()

        def g_issue(par, k, b):
            pltpu.async_copy(xs_hbm.at[src_v.at[par, k]], rows_v.at[b], gsem[b])

        def g_wait(par, k, b):
            pltpu.make_async_copy(
                xs_hbm.at[src_v.at[par, k]], rows_v.at[b], gsem[b]).wait()

        def s_issue(par, k, b):
            pltpu.async_copy(rows_v.at[b], acc.at[dst_v.at[par, k]],
                             ssem[b], add=True)

        def s_wait(par, k, b):
            pltpu.make_async_copy(
                rows_v.at[b], acc.at[dst_v.at[par, k]], ssem[b]).wait()

        def do_iter(it, first):
            off = it * (2 * SNBC)
            pltpu.sync_copy(src_hbm.at[wid, pl.ds(off, SNBC)], src_v.at[0])
            pltpu.sync_copy(dst_hbm.at[wid, pl.ds(off, SNBC)], dst_v.at[0])
            pltpu.sync_copy(src_hbm.at[wid, pl.ds(off + SNBC, SNBC)], src_v.at[1])
            pltpu.sync_copy(dst_hbm.at[wid, pl.ds(off + SNBC, SNBC)], dst_v.at[1])
            for j in (0, 1):
                g_issue(0, j, j)
            for j in range(2 * SNBC):
                par, k, b = j // SNBC, j % SNBC, j % 4
                if j < 2 * SNBC - 2:
                    jj = j + 2
                    if not (first and j < 2):
                        s_wait(jj // SNBC, jj % SNBC, jj % 4)
                    g_issue(jj // SNBC, jj % SNBC, jj % 4)
                g_wait(par, k, b)
                s_issue(par, k, b)

        do_iter(0, True)

        @pl.loop(1, SNPH // 2)
        def _(it):
            do_iter(it, False)

        s_wait(1, SNBC - 2, (2 * SNBC - 2) % 4)
        s_wait(1, SNBC - 1, (2 * SNBC - 1) % 4)

        plsc.subcore_barrier()
        for t in range(RPT // SBLK):
            pltpu.sync_copy(acc.at[pl.ds(base + t * SBLK, SBLK)], rows_v.at[0])
            pltpu.sync_copy(rows_v.at[0],
                            out_hbm.at[cid, pl.ds(base + t * SBLK, SBLK)])
        tail = base + (RPT // SBLK) * SBLK
        pltpu.sync_copy(acc.at[pl.ds(tail, rem)], rows_v.at[0, pl.ds(0, rem)])
        pltpu.sync_copy(rows_v.at[0, pl.ds(0, rem)],
                        out_hbm.at[cid, pl.ds(tail, rem)])

    pl.run_scoped(
        body,
        pltpu.VMEM((2, SNBC, SBLK), jnp.int32),
        pltpu.VMEM((2, SNBC, SBLK), jnp.int32),
        pltpu.VMEM((4, SBLK, C), jnp.float32),
        pltpu.SemaphoreType.DMA,
        pltpu.SemaphoreType.DMA,
        pltpu.SemaphoreType.DMA,
        pltpu.SemaphoreType.DMA,
        pltpu.SemaphoreType.DMA,
        pltpu.SemaphoreType.DMA,
        pltpu.SemaphoreType.DMA,
        pltpu.SemaphoreType.DMA,
    )


def _prep_body(dacc_ref, x_ref, xs_ref, dinv_ref):
    d = dacc_ref[0] + dacc_ref[1]
    dinv = lax.rsqrt(d[:, 0:1] + 1.0)
    xs_ref[...] = x_ref[...] * dinv
    dinv_ref[...] = dinv


_prep = pl.pallas_call(
    _prep_body,
    grid=(GRID,),
    in_specs=[
        pl.BlockSpec((NC, R, DEGW), lambda i: (0, i, 0)),
        pl.BlockSpec((R, C), lambda i: (i, 0)),
    ],
    out_specs=[
        pl.BlockSpec((R, C), lambda i: (i, 0)),
        pl.BlockSpec((R, 1), lambda i: (i, 0)),
    ],
    out_shape=[
        jax.ShapeDtypeStruct((NPAD, C), jnp.float32),
        jax.ShapeDtypeStruct((NPAD, 1), jnp.float32),
    ],
)


def _mid_body(s_ref, xs_ref, dinv_ref, w_ref, b_ref, hs_ref):
    g = (s_ref[0] + s_ref[1] + xs_ref[...]) * dinv_ref[...]
    h = jnp.dot(g, w_ref[...], preferred_element_type=jnp.float32) + b_ref[...]
    hs_ref[...] = jnp.maximum(h, 0.0) * dinv_ref[...]


_mid = pl.pallas_call(
    _mid_body,
    grid=(GRID,),
    in_specs=[
        pl.BlockSpec((NC, R, C), lambda i: (0, i, 0)),
        pl.BlockSpec((R, C), lambda i: (i, 0)),
        pl.BlockSpec((R, 1), lambda i: (i, 0)),
        pl.BlockSpec((C, C), lambda i: (0, 0)),
        pl.BlockSpec((1, C), lambda i: (0, 0)),
    ],
    out_specs=pl.BlockSpec((R, C), lambda i: (i, 0)),
    out_shape=jax.ShapeDtypeStruct((NPAD, C), jnp.float32),
)


def _out_body(s_ref, hs_ref, dinv_ref, wmu_ref, bmu_ref, wls_ref, bls_ref,
              mu_ref, ls_ref):
    g = (s_ref[0] + s_ref[1] + hs_ref[...]) * dinv_ref[...]
    mu_ref[...] = jnp.dot(g, wmu_ref[...], preferred_element_type=jnp.float32) + bmu_ref[...]
    ls_ref[...] = jnp.dot(g, wls_ref[...], preferred_element_type=jnp.float32) + bls_ref[...]


_outk = pl.pallas_call(
    _out_body,
    grid=(GRID,),
    in_specs=[
        pl.BlockSpec((NC, R, C), lambda i: (0, i, 0)),
        pl.BlockSpec((R, C), lambda i: (i, 0)),
        pl.BlockSpec((R, 1), lambda i: (i, 0)),
        pl.BlockSpec((C, OC), lambda i: (0, 0)),
        pl.BlockSpec((1, OC), lambda i: (0, 0)),
        pl.BlockSpec((C, OC), lambda i: (0, 0)),
        pl.BlockSpec((1, OC), lambda i: (0, 0)),
    ],
    out_specs=[
        pl.BlockSpec((R, OC), lambda i: (i, 0)),
        pl.BlockSpec((R, OC), lambda i: (i, 0)),
    ],
    out_shape=[
        jax.ShapeDtypeStruct((NPAD, OC), jnp.float32),
        jax.ShapeDtypeStruct((NPAD, OC), jnp.float32),
    ],
)


def kernel(x, edge_index, W1, b1, W_mu, b_mu, W_ls, b_ls):
    ei = edge_index.astype(jnp.int32)
    src, dst = ei[0], ei[1]
    fill = jnp.arange(E_PAD - E, dtype=jnp.int32)
    src_p = jnp.concatenate([src, fill % N])
    dst_p = jnp.concatenate([dst, N + fill % (NPAD - N)])
    src_s = src_p.reshape(NW, SNB, SBLK)
    dst_s = dst_p.reshape(NW, SNB, SBLK)
    dst_d = dst_p.reshape(NW, NB, BLK)
    xp = jnp.pad(x, ((0, NPAD - N), (0, 0)))

    dacc = _deg_kernel(dst_d)
    xs, dinv = _prep(dacc, xp)
    s0 = _scatter_kernel(xs, src_s, dst_s)
    hs = _mid(s0, xs, dinv, W1, b1.reshape(1, C))
    s1 = _scatter_kernel(hs, src_s, dst_s)
    mu, ls = _outk(s1, hs, dinv, W_mu, b_mu.reshape(1, OC),
                   W_ls, b_ls.reshape(1, OC))
    return (mu[:N], ls[:N])


# speedup vs baseline: 1.0973x; 1.0973x over previous
import functools

import jax
import jax.numpy as jnp
from jax import lax
from jax.experimental import pallas as pl
from jax.experimental.pallas import tpu as pltpu
from jax.experimental.pallas import tpu_sc as plsc

N = 10000
C = 128
OC = 64
E = 320000

NC, NS = 2, 16
NW = NC * NS
BLK = 128
NB = 80
NBC = 8
NPH = NB // NBC
EPW = NB * BLK
E_PAD = EPW * NW
NPAD = 10112
RPT = NPAD // NS
DEGW = 16

R = 1000
GRID = N // R


def _sc_mesh():
    return plsc.VectorSubcoreMesh(
        core_axis_name="c", subcore_axis_name="s",
        num_cores=NC, num_subcores=NS)


@functools.partial(
    pl.kernel,
    out_type=jax.ShapeDtypeStruct((NC, NPAD, DEGW), jnp.float32),
    mesh=_sc_mesh(),
    scratch_types=[
        pltpu.VMEM_SHARED((NPAD, DEGW), jnp.float32),
    ],
)
def _deg_kernel(dst_hbm, out_hbm, acc):
    cid = lax.axis_index("c")
    sid = lax.axis_index("s")
    wid = cid * NS + sid
    one = jnp.ones((16,), jnp.float32)
    zero = jnp.zeros((16,), jnp.float32)

    def body(dst_v, ones_v, zs_v, dsem):
        @pl.loop(0, BLK)
        def _(i):
            ones_v[i, :] = one

        @pl.loop(0, RPT)
        def _(i):
            zs_v[i, :] = zero

        base = sid * RPT
        pltpu.sync_copy(zs_v, acc.at[pl.ds(base, RPT)])
        pltpu.sync_copy(dst_hbm.at[wid], dst_v)
        plsc.subcore_barrier()

        @pl.loop(0, NB)
        def _(j):
            pltpu.sync_copy(ones_v, acc.at[dst_v.at[j]], add=True)

        plsc.subcore_barrier()
        pltpu.sync_copy(acc.at[pl.ds(base, RPT)],
                        out_hbm.at[cid, pl.ds(base, RPT)])

    pl.run_scoped(
        body,
        pltpu.VMEM((NB, BLK), jnp.int32),
        pltpu.VMEM((BLK, DEGW), jnp.float32),
        pltpu.VMEM((RPT, DEGW), jnp.float32),
        pltpu.SemaphoreType.DMA,
    )


@functools.partial(
    pl.kernel,
    out_type=jax.ShapeDtypeStruct((NC, NPAD, C), jnp.float32),
    mesh=_sc_mesh(),
    scratch_types=[
        pltpu.VMEM_SHARED((NPAD, C), jnp.float32),
    ],
)
def _scatter_kernel(xs_hbm, src_hbm, dst_hbm, out_hbm, acc):
    cid = lax.axis_index("c")
    sid = lax.axis_index("s")
    wid = cid * NS + sid
    zero = jnp.zeros((16,), jnp.float32)
    base = sid * RPT

    def body(src_v, dst_v, rows_v, sem0, sem1, osem0, osem1):
        osems = (osem0, osem1)
        @pl.loop(0, BLK)
        def _(i):
            for j in range(C // 16):
                rows_v[0, i, pl.ds(j * 16, 16)] = zero

        rem = RPT - (RPT // BLK) * BLK
        for t in range(RPT // BLK):
            pltpu.async_copy(rows_v.at[0], acc.at[pl.ds(base + t * BLK, BLK)],
                             sem0)
        pltpu.async_copy(rows_v.at[0, pl.ds(0, rem)],
                         acc.at[pl.ds(base + (RPT // BLK) * BLK, rem)], sem0)
        for t in range(RPT // BLK):
            pltpu.make_async_copy(rows_v.at[0],
                                  acc.at[pl.ds(base + t * BLK, BLK)],
                                  sem0).wait()
        pltpu.make_async_copy(rows_v.at[0, pl.ds(0, rem)],
                              acc.at[pl.ds(base + (RPT // BLK) * BLK, rem)],
                              sem0).wait()
        plsc.subcore_barrier()

        @pl.loop(0, NPH)
        def _(p):
            pltpu.sync_copy(src_hbm.at[wid, pl.ds(p * NBC, NBC)], src_v)
            pltpu.sync_copy(dst_hbm.at[wid, pl.ds(p * NBC, NBC)], dst_v)
            pltpu.async_copy(xs_hbm.at[src_v.at[0]], rows_v.at[0], sem0)
            pltpu.async_copy(xs_hbm.at[src_v.at[1]], rows_v.at[1], sem1)

            @pl.loop(0, NBC // 2 - 1)
            def _(i):
                j = 2 * i
                pltpu.make_async_copy(xs_hbm.at[src_v.at[j]], rows_v.at[0], sem0).wait()
                pltpu.sync_copy(rows_v.at[0], acc.at[dst_v.at[j]], add=True)
                pltpu.async_copy(xs_hbm.at[src_v.at[j + 2]], rows_v.at[0], sem0)
                pltpu.make_async_copy(xs_hbm.at[src_v.at[j + 1]], rows_v.at[1], sem1).wait()
                pltpu.sync_copy(rows_v.at[1], acc.at[dst_v.at[j + 1]], add=True)
                pltpu.async_copy(xs_hbm.at[src_v.at[j + 3]], rows_v.at[1], sem1)

            pltpu.make_async_copy(xs_hbm.at[src_v.at[NBC - 2]], rows_v.at[0], sem0).wait()
            pltpu.sync_copy(rows_v.at[0], acc.at[dst_v.at[NBC - 2]], add=True)
            pltpu.make_async_copy(xs_hbm.at[src_v.at[NBC - 1]], rows_v.at[1], sem1).wait()
            pltpu.sync_copy(rows_v.at[1], acc.at[dst_v.at[NBC - 1]], add=True)

        plsc.subcore_barrier()
        tail = base + (RPT // BLK) * BLK
        nfull = RPT // BLK
        sems = (sem0, sem1)

        def cp_in(t, b):
            if t < nfull:
                return (acc.at[pl.ds(base + t * BLK, BLK)], rows_v.at[b])
            return (acc.at[pl.ds(tail, rem)], rows_v.at[b, pl.ds(0, rem)])

        def cp_out(t, b):
            if t < nfull:
                return (rows_v.at[b], out_hbm.at[cid, pl.ds(base + t * BLK, BLK)])
            return (rows_v.at[b, pl.ds(0, rem)],
                    out_hbm.at[cid, pl.ds(tail, rem)])

        nchunk = nfull + 1
        for t in (0, 1):
            sin, din = cp_in(t, t)
            pltpu.async_copy(sin, din, sems[t])
        for t in range(nchunk):
            b = t % 2
            sin, din = cp_in(t, b)
            pltpu.make_async_copy(sin, din, sems[b]).wait()
            so_, do_ = cp_out(t, b)
            pltpu.async_copy(so_, do_, osems[b])
            if t + 2 < nchunk:
                pltpu.make_async_copy(so_, do_, osems[b]).wait()
                sin, din = cp_in(t + 2, b)
                pltpu.async_copy(sin, din, sems[b])
        for t in (nchunk - 2, nchunk - 1):
            b = t % 2
            so_, do_ = cp_out(t, b)
            pltpu.make_async_copy(so_, do_, osems[b]).wait()

    pl.run_scoped(
        body,
        pltpu.VMEM((NBC, BLK), jnp.int32),
        pltpu.VMEM((NBC, BLK), jnp.int32),
        pltpu.VMEM((2, BLK, C), jnp.float32),
        pltpu.SemaphoreType.DMA,
        pltpu.SemaphoreType.DMA,
        pltpu.SemaphoreType.DMA,
        pltpu.SemaphoreType.DMA,
    )


def _prep_body(dacc_ref, x_ref, xs_ref, dinv_ref):
    d = dacc_ref[0] + dacc_ref[1]
    dinv = lax.rsqrt(d[:, 0:1] + 1.0)
    xs_ref[...] = x_ref[...] * dinv
    dinv_ref[...] = dinv


_prep = pl.pallas_call(
    _prep_body,
    grid=(GRID,),
    in_specs=[
        pl.BlockSpec((NC, R, DEGW), lambda i: (0, i, 0)),
        pl.BlockSpec((R, C), lambda i: (i, 0)),
    ],
    out_specs=[
        pl.BlockSpec((R, C), lambda i: (i, 0)),
        pl.BlockSpec((R, 1), lambda i: (i, 0)),
    ],
    out_shape=[
        jax.ShapeDtypeStruct((N, C), jnp.float32),
        jax.ShapeDtypeStruct((N, 1), jnp.float32),
    ],
)


def _mid_body(s_ref, xs_ref, dinv_ref, w_ref, b_ref, hs_ref):
    g = (s_ref[0] + s_ref[1] + xs_ref[...]) * dinv_ref[...]
    h = jnp.dot(g, w_ref[...], preferred_element_type=jnp.float32) + b_ref[...]
    hs_ref[...] = jnp.maximum(h, 0.0) * dinv_ref[...]


_mid = pl.pallas_call(
    _mid_body,
    grid=(GRID,),
    in_specs=[
        pl.BlockSpec((NC, R, C), lambda i: (0, i, 0)),
        pl.BlockSpec((R, C), lambda i: (i, 0)),
        pl.BlockSpec((R, 1), lambda i: (i, 0)),
        pl.BlockSpec((C, C), lambda i: (0, 0)),
        pl.BlockSpec((1, C), lambda i: (0, 0)),
    ],
    out_specs=pl.BlockSpec((R, C), lambda i: (i, 0)),
    out_shape=jax.ShapeDtypeStruct((N, C), jnp.float32),
)


def _out_body(s_ref, hs_ref, dinv_ref, wmu_ref, bmu_ref, wls_ref, bls_ref,
              mu_ref, ls_ref):
    g = (s_ref[0] + s_ref[1] + hs_ref[...]) * dinv_ref[...]
    mu_ref[...] = jnp.dot(g, wmu_ref[...], preferred_element_type=jnp.float32) + bmu_ref[...]
    ls_ref[...] = jnp.dot(g, wls_ref[...], preferred_element_type=jnp.float32) + bls_ref[...]


_outk = pl.pallas_call(
    _out_body,
    grid=(GRID,),
    in_specs=[
        pl.BlockSpec((NC, R, C), lambda i: (0, i, 0)),
        pl.BlockSpec((R, C), lambda i: (i, 0)),
        pl.BlockSpec((R, 1), lambda i: (i, 0)),
        pl.BlockSpec((C, OC), lambda i: (0, 0)),
        pl.BlockSpec((1, OC), lambda i: (0, 0)),
        pl.BlockSpec((C, OC), lambda i: (0, 0)),
        pl.BlockSpec((1, OC), lambda i: (0, 0)),
    ],
    out_specs=[
        pl.BlockSpec((R, OC), lambda i: (i, 0)),
        pl.BlockSpec((R, OC), lambda i: (i, 0)),
    ],
    out_shape=[
        jax.ShapeDtypeStruct((N, OC), jnp.float32),
        jax.ShapeDtypeStruct((N, OC), jnp.float32),
    ],
)


def kernel(x, edge_index, W1, b1, W_mu, b_mu, W_ls, b_ls):
    ei = edge_index.astype(jnp.int32)
    src, dst = ei[0], ei[1]
    fill = jnp.arange(E_PAD - E, dtype=jnp.int32)
    src_p = jnp.concatenate([src, fill % N]).reshape(NW, NB, BLK)
    dst_p = jnp.concatenate([dst, N + fill % (NPAD - N)]).reshape(NW, NB, BLK)
    dacc = _deg_kernel(dst_p)
    xs, dinv = _prep(dacc, x)
    s0 = _scatter_kernel(xs, src_p, dst_p)
    hs = _mid(s0, xs, dinv, W1, b1.reshape(1, C))
    s1 = _scatter_kernel(hs, src_p, dst_p)
    mu, ls = _outk(s1, hs, dinv, W_mu, b_mu.reshape(1, OC),
                   W_ls, b_ls.reshape(1, OC))
    return (mu, ls)
